# Initial kernel scaffold; baseline (speedup 1.0000x reference)
import jax
import jax.numpy as jnp
from jax.experimental import pallas as pl


def kernel(atomic_numbers, pos, edge_index, centers, widths, node_emb, layers, readout, atomic_e):
    raise NotImplementedError("write your pallas kernel here")


# speedup vs baseline: 1.9319x; 1.9319x over previous
import functools
import math

import jax
import jax.numpy as jnp
from jax import lax
from jax.experimental import pallas as pl
from jax.experimental.pallas import tpu as pltpu
from jax.experimental.pallas import tpu_sc as plsc

N = 50000
E = 800000
H = 64
NB = 8
CUTOFF = 5.0
NCORES = 2
NSUB = 16
CH = 128
E_PAD = 802816
HALF = 25000
SPAD = 26624
TRASH = 25600
ROWS_PER_TILE = 1664
NCHUNK_B = E_PAD // (32 * CH)
NCHUNK_D = E_PAD // (16 * CH)
RB_E = 2048
RB_N = 2000

_f32 = jnp.float32
_i32 = jnp.int32


def _silu(x):
    return x * jax.nn.sigmoid(x)


def _edge_vec_sc(pos_pad, col_pad, row_pad):
    mesh = plsc.VectorSubcoreMesh(core_axis_name="c", subcore_axis_name="s")

    @functools.partial(
        pl.kernel,
        out_type=jax.ShapeDtypeStruct((E_PAD, 16), _f32),
        mesh=mesh,
        scratch_types=[
            pltpu.VMEM((CH,), _i32),
            pltpu.VMEM((CH,), _i32),
            pltpu.VMEM((CH, 16), _f32),
            pltpu.VMEM((CH, 16), _f32),
            pltpu.SemaphoreType.DMA,
            pltpu.SemaphoreType.DMA,
        ],
        compiler_params=pltpu.CompilerParams(use_tc_tiling_on_sc=False),
    )
    def body(pos_hbm, col_hbm, row_hbm, ev_hbm, colv, rowv, pv, pr, sem1, sem2):
        c = lax.axis_index("c")
        s = lax.axis_index("s")
        wid = c * NSUB + s

        def chunk(g, carry):
            base = (wid * NCHUNK_B + g) * CH
            pltpu.sync_copy(col_hbm.at[pl.ds(base, CH)], colv)
            pltpu.sync_copy(row_hbm.at[pl.ds(base, CH)], rowv)
            cp1 = pltpu.async_copy(pos_hbm.at[colv], pv, sem1)
            cp2 = pltpu.async_copy(pos_hbm.at[rowv], pr, sem2)
            cp1.wait()
            cp2.wait()

            def sub_row(j, carry2):
                pv[j, pl.ds(0, 16)] = pv[j, pl.ds(0, 16)] - pr[j, pl.ds(0, 16)]
                return carry2

            lax.fori_loop(0, CH, sub_row, 0)
            pltpu.sync_copy(pv, ev_hbm.at[pl.ds(base, CH)])
            return carry

        lax.fori_loop(0, NCHUNK_B, chunk, 0)

    return body(pos_pad, col_pad, row_pad)


def _edge_w_tc(ev, a1s, b1s, a2s, b2s, centers_row, widths_row):
    grid = (E_PAD // RB_E,)

    def body(ev_ref, a1_ref, b1_ref, a2_ref, b2_ref, c_ref, wd_ref,
             w0_ref, w1_ref, w2_ref):
        v = ev_ref[...]
        sq = v * v
        len2 = sq[:, 0:1] + sq[:, 1:2] + sq[:, 2:3]
        elen = jnp.sqrt(len2)
        xc = elen * (math.pi / CUTOFF)
        cut = 0.5 * (jnp.cos(xc) + 1.0)
        cut = cut * (elen < CUTOFF).astype(_f32)
        diff = elen - c_ref[...]
        basis = jnp.exp(-0.5 * (diff * wd_ref[...]) ** 2)
        rbf = basis * cut
        outs = (w0_ref, w1_ref, w2_ref)
        for l in range(3):
            h1 = _silu(jnp.dot(rbf, a1_ref[l], preferred_element_type=_f32)
                       + b1_ref[l, :][None, :])
            outs[l][...] = (jnp.dot(h1, a2_ref[l], preferred_element_type=_f32)
                            + b2_ref[l, :][None, :])

    full3 = lambda shp: pl.BlockSpec(shp, lambda i: (0,) * len(shp))
    return pl.pallas_call(
        body,
        grid=grid,
        in_specs=[
            pl.BlockSpec((RB_E, 16), lambda i: (i, 0)),
            full3((3, NB, H)), full3((3, H)), full3((3, H, H)), full3((3, H)),
            full3((1, NB)), full3((1, NB)),
        ],
        out_specs=[pl.BlockSpec((RB_E, H), lambda i: (i, 0))] * 3,
        out_shape=[jax.ShapeDtypeStruct((E_PAD, H), _f32)] * 3,
    )(ev, a1s, b1s, a2s, b2s, centers_row, widths_row)


def _gather_mul_scatter_sc(feats, w, col_pad, row_d):
    mesh = plsc.VectorSubcoreMesh(core_axis_name="c", subcore_axis_name="s")

    @functools.partial(
        pl.kernel,
        out_type=jax.ShapeDtypeStruct((N, H), _f32),
        mesh=mesh,
        scratch_types=[
            pltpu.VMEM((CH,), _i32),
            pltpu.VMEM((CH,), _i32),
            pltpu.VMEM((CH,), _i32),
            pltpu.VMEM((CH, H), _f32),
            pltpu.VMEM((CH, H), _f32),
            pltpu.VMEM_SHARED((SPAD, H), _f32),
            pltpu.SemaphoreType.DMA,
        ],
        compiler_params=pltpu.CompilerParams(use_tc_tiling_on_sc=False),
    )
    def body(feats_hbm, w_hbm, col_hbm, row_hbm, out_hbm,
             colv, rowv, radjv, fv, wv, aggS, sem):
        c = lax.axis_index("c")
        s = lax.axis_index("s")
        c_base = c * HALF

        def zrow(j, carry):
            for q in range(H // 16):
                wv[j, pl.ds(q * 16, 16)] = jnp.zeros((16,), _f32)
            return carry

        lax.fori_loop(0, CH, zrow, 0)

        def zcp(k, carry):
            pltpu.sync_copy(wv, aggS.at[pl.ds(s * ROWS_PER_TILE + k * CH, CH)])
            return carry

        lax.fori_loop(0, ROWS_PER_TILE // CH, zcp, 0)
        plsc.subcore_barrier()

        def chunk(g, carry):
            base = (s * NCHUNK_D + g) * CH
            pltpu.sync_copy(col_hbm.at[pl.ds(base, CH)], colv)
            pltpu.sync_copy(row_hbm.at[pl.ds(base, CH)], rowv)
            gcp = pltpu.async_copy(feats_hbm.at[colv], fv, sem)
            pltpu.sync_copy(w_hbm.at[pl.ds(base, CH)], wv)
            gcp.wait()

            def radj_i(i, carry2):
                r = rowv[pl.ds(i * 16, 16)]
                inr = (r >= c_base) & (r < c_base + HALF)
                radjv[pl.ds(i * 16, 16)] = jnp.where(inr, r - c_base, TRASH)
                return carry2

            lax.fori_loop(0, CH // 16, radj_i, 0)

            def mul_j(j, carry2):
                for q in range(H // 16):
                    fv[j, pl.ds(q * 16, 16)] = (fv[j, pl.ds(q * 16, 16)]
                                                * wv[j, pl.ds(q * 16, 16)])
                return carry2

            lax.fori_loop(0, CH, mul_j, 0)
            pltpu.sync_copy(fv, aggS.at[radjv], add=True)
            return carry

        lax.fori_loop(0, NCHUNK_D, chunk, 0)
        plsc.subcore_barrier()

        @pl.when(s < NSUB - 1)
        def _():
            pltpu.sync_copy(
                aggS.at[pl.ds(s * ROWS_PER_TILE, ROWS_PER_TILE)],
                out_hbm.at[pl.ds(c_base + s * ROWS_PER_TILE, ROWS_PER_TILE)])

        @pl.when(s == NSUB - 1)
        def _():
            tail = HALF - (NSUB - 1) * ROWS_PER_TILE
            pltpu.sync_copy(
                aggS.at[pl.ds((NSUB - 1) * ROWS_PER_TILE, tail)],
                out_hbm.at[pl.ds(c_base + (NSUB - 1) * ROWS_PER_TILE, tail)])

    return body(feats, w, col_pad, row_d)


def _embed_tc(an3, emb_pad):
    def body(an_ref, emb_ref, out_ref):
        ids = an_ref[0, 0, :]
        onehot = (ids[:, None] ==
                  lax.broadcasted_iota(_i32, (RB_N, 128), 1)).astype(_f32)
        out_ref[...] = jnp.dot(onehot, emb_ref[...], preferred_element_type=_f32)

    return pl.pallas_call(
        body,
        grid=(N // RB_N,),
        in_specs=[
            pl.BlockSpec((1, 1, RB_N), lambda i: (i, 0, 0)),
            pl.BlockSpec((128, H), lambda i: (0, 0)),
        ],
        out_specs=pl.BlockSpec((RB_N, H), lambda i: (i, 0)),
        out_shape=jax.ShapeDtypeStruct((N, H), _f32),
    )(an3, emb_pad)


def _node_dense_tc(feats, agg, m1, m2, bc, u1, ub1, u2, ub2, lng, lnb):
    def body(f_ref, a_ref, m1_ref, m2_ref, bc_ref, u1_ref, ub1_ref,
             u2_ref, ub2_ref, g_ref, b_ref, out_ref):
        f = f_ref[...]
        conv = (jnp.dot(f, m1_ref[...], preferred_element_type=_f32)
                + jnp.dot(a_ref[...], m2_ref[...], preferred_element_type=_f32)
                + bc_ref[...])
        hid = _silu(jnp.dot(conv, u1_ref[...], preferred_element_type=_f32)
                    + ub1_ref[...])
        upd = jnp.dot(hid, u2_ref[...], preferred_element_type=_f32) + ub2_ref[...]
        h = f + upd
        mu = jnp.mean(h, axis=-1, keepdims=True)
        var = jnp.mean((h - mu) ** 2, axis=-1, keepdims=True)
        out_ref[...] = ((h - mu) * lax.rsqrt(var + 1e-5) * g_ref[...]
                        + b_ref[...])

    full = lambda shp: pl.BlockSpec(shp, lambda i: (0,) * len(shp))
    return pl.pallas_call(
        body,
        grid=(N // RB_N,),
        in_specs=[
            pl.BlockSpec((RB_N, H), lambda i: (i, 0)),
            pl.BlockSpec((RB_N, H), lambda i: (i, 0)),
            full((H, H)), full((H, H)), full((1, H)),
            full((H, 2 * H)), full((1, 2 * H)),
            full((2 * H, H)), full((1, H)),
            full((1, H)), full((1, H)),
        ],
        out_specs=pl.BlockSpec((RB_N, H), lambda i: (i, 0)),
        out_shape=jax.ShapeDtypeStruct((N, H), _f32),
    )(feats, agg, m1, m2, bc, u1, ub1, u2, ub2, lng, lnb)


def _readout_tc(feats, an3, r1t, rb1, r2t, rb2, r3row, rb3, ae_row):
    def body(f_ref, an_ref, r1_ref, rb1_ref, r2_ref, rb2_ref, r3_ref,
             rb3_ref, ae_ref, out_ref):
        f = f_ref[...]
        e = _silu(jnp.dot(f, r1_ref[...], preferred_element_type=_f32)
                  + rb1_ref[...])
        e = _silu(jnp.dot(e, r2_ref[...], preferred_element_type=_f32)
                  + rb2_ref[...])
        ev = jnp.sum(e * r3_ref[...], axis=-1) + rb3_ref[...][0, 0]
        ids = an_ref[0, 0, :]
        onehot = (ids[:, None] ==
                  lax.broadcasted_iota(_i32, (RB_N, 128), 1)).astype(_f32)
        aev = jnp.sum(onehot * ae_ref[...], axis=-1)
        ssum = jnp.sum(ev + aev)

        @pl.when(pl.program_id(0) == 0)
        def _():
            out_ref[...] = jnp.zeros_like(out_ref)

        out_ref[...] += jnp.reshape(ssum, (1, 1))

    full = lambda shp: pl.BlockSpec(shp, lambda i: (0,) * len(shp))
    return pl.pallas_call(
        body,
        grid=(N // RB_N,),
        in_specs=[
            pl.BlockSpec((RB_N, H), lambda i: (i, 0)),
            pl.BlockSpec((1, 1, RB_N), lambda i: (i, 0, 0)),
            full((H, H)), full((1, H)),
            full((H, H // 2)), full((1, H // 2)),
            full((1, H // 2)), full((1, 1)), full((1, 128)),
        ],
        out_specs=pl.BlockSpec((1, 1), lambda i: (0, 0)),
        out_shape=jax.ShapeDtypeStruct((1, 1), _f32),
    )(feats, an3, r1t, rb1, r2t, rb2, r3row, rb3, ae_row)


def kernel(atomic_numbers, pos, edge_index, centers, widths, node_emb,
           layers, readout, atomic_e):
    an = atomic_numbers.astype(_i32)
    an3 = an.reshape(N // RB_N, 1, RB_N)
    row = edge_index[0].astype(_i32)
    col = edge_index[1].astype(_i32)
    npad = E_PAD - E
    col_pad = jnp.concatenate([col, jnp.zeros((npad,), _i32)])
    row_b = jnp.concatenate([row, jnp.zeros((npad,), _i32)])
    row_d = jnp.concatenate([row, jnp.full((npad,), -1, _i32)])
    pos_pad = jnp.zeros((N, 16), _f32).at[:, :3].set(pos)
    emb_pad = jnp.zeros((128, H), _f32).at[:100].set(node_emb)
    centers_row = centers.reshape(1, NB).astype(_f32)
    inv_w = (1.0 / jnp.clip(widths, 0.1, None)).reshape(1, NB).astype(_f32)

    a1s = jnp.stack([jnp.transpose(p['rn_W1']) for p in layers])
    b1s = jnp.stack([p['rn_b1'] for p in layers])
    a2s = jnp.stack([jnp.transpose(p['rn_W2']) for p in layers])
    b2s = jnp.stack([p['rn_b2'] for p in layers])

    node_w = []
    for p in layers:
        cp_t = jnp.transpose(p['cp_W'])
        cp_top, cp_bot = cp_t[:H], cp_t[H:]
        m1 = jnp.transpose(p['si_W']) @ cp_top
        bc = (p['si_b'] @ cp_top + p['cp_b']).reshape(1, H)
        node_w.append(dict(
            m1=m1, m2=cp_bot, bc=bc,
            u1=jnp.transpose(p['u_W1']), ub1=p['u_b1'].reshape(1, 2 * H),
            u2=jnp.transpose(p['u_W2']), ub2=p['u_b2'].reshape(1, H),
            lng=p['ln_g'].reshape(1, H), lnb=p['ln_b'].reshape(1, H)))

    r1t = jnp.transpose(readout['W1'])
    rb1 = readout['b1'].reshape(1, H)
    r2t = jnp.transpose(readout['W2'])
    rb2 = readout['b2'].reshape(1, H // 2)
    r3row = readout['W3'].reshape(1, H // 2)
    rb3 = readout['b3'].reshape(1, 1)
    ae_row = jnp.zeros((1, 128), _f32).at[0, :100].set(atomic_e[:, 0])

    ev = _edge_vec_sc(pos_pad, col_pad, row_b)
    w0, w1, w2 = _edge_w_tc(ev, a1s, b1s, a2s, b2s, centers_row, inv_w)
    feats = _embed_tc(an3, emb_pad)
    for li, wl in enumerate((w0, w1, w2)):
        agg = _gather_mul_scatter_sc(feats, wl, col_pad, row_d)
        nw = node_w[li]
        feats = _node_dense_tc(feats, agg, nw['m1'], nw['m2'], nw['bc'],
                               nw['u1'], nw['ub1'], nw['u2'], nw['ub2'],
                               nw['lng'], nw['lnb'])
    out = _readout_tc(feats, an3, r1t, rb1, r2t, rb2, r3row, rb3, ae_row)
    return out[0, 0]
